# baseline (speedup 1.0000x reference)
import functools

import jax
import jax.numpy as jnp
from jax import lax
from jax.experimental import pallas as pl
from jax.experimental.pallas import tpu as pltpu
from jax.experimental.pallas import tpu_sc as plsc

_NC, _NS, _L = 2, 16, 16
_NW = _NC * _NS
_N = 16384 * 128
_PER_W = _N // _NW
_CHUNK = 8192
_NCHUNK = _PER_W // _CHUNK
_VEC = 8
_CN = 16384 * 128 * 3
_PER_WC = _CN // _NW


_GATHER_DNUMS = lax.GatherDimensionNumbers(
    offset_dims=(), collapsed_slice_dims=(0,), start_index_map=(0,))


def _wrap_table16():
    i = lax.iota(jnp.int32, 16)
    return jnp.where(i == jnp.int32(10), jnp.int32(-1),
                     jnp.where(i >= jnp.int32(11), i - jnp.int32(12),
                               i - jnp.int32(1)))


def _map_vec(x, tbl):
    idx = lax.bitwise_and(x, jnp.int32(15))
    return lax.gather(tbl, idx[:, None], _GATHER_DNUMS, (1,),
                      mode=lax.GatherScatterMode.PROMISE_IN_BOUNDS)


@functools.partial(
    pl.kernel,
    mesh=plsc.VectorSubcoreMesh(core_axis_name="c", subcore_axis_name="s"),
    out_type=(
        jax.ShapeDtypeStruct((_N,), jnp.int32),
        jax.ShapeDtypeStruct((_CN,), jnp.float32),
    ),
    scratch_types=[
        pltpu.VMEM((2, _CHUNK), jnp.int32),
        pltpu.VMEM((2, _CHUNK), jnp.int32),
        pltpu.SemaphoreType.DMA,
        pltpu.SemaphoreType.DMA,
        pltpu.SemaphoreType.DMA,
        pltpu.SemaphoreType.DMA,
        pltpu.SemaphoreType.DMA,
    ],
)
def _convert(sp_hbm, coord_hbm, out_hbm, coord_out_hbm,
             ibuf, obuf, si0, si1, so0, so1, sc):
    wid = lax.axis_index("s") * _NC + lax.axis_index("c")
    base = wid * _PER_W
    tbl = _wrap_table16()
    sem_in, sem_out = (si0, si1), (so0, so1)

    cbase = wid * _PER_WC
    coord_copy = pltpu.make_async_copy(
        coord_hbm.at[pl.ds(cbase, _PER_WC)],
        coord_out_hbm.at[pl.ds(cbase, _PER_WC)], sc)
    coord_copy.start()

    def _start_in(g):
        slot = g % 2
        pltpu.async_copy(sp_hbm.at[pl.ds(base + g * _CHUNK, _CHUNK)],
                         ibuf.at[slot], sem_in[slot])

    def _wait_in(g):
        slot = g % 2
        pltpu.make_async_copy(sp_hbm.at[pl.ds(0, _CHUNK)], ibuf.at[slot],
                              sem_in[slot]).wait()

    def _compute(slot):
        def body(i, _):
            b0 = i * (_L * _VEC)
            offs = [pl.multiple_of(b0 + k * _L, _L) for k in range(_VEC)]
            xs = [ibuf[slot, pl.ds(o, _L)] for o in offs]
            ys = [_map_vec(x, tbl) for x in xs]
            for o, y in zip(offs, ys):
                obuf[slot, pl.ds(o, _L)] = y
            return 0
        lax.fori_loop(0, _CHUNK // (_L * _VEC), body, 0)

    def _start_out(g):
        slot = g % 2
        pltpu.async_copy(obuf.at[slot],
                         out_hbm.at[pl.ds(base + g * _CHUNK, _CHUNK)],
                         sem_out[slot])

    def _wait_out(g):
        slot = g % 2
        pltpu.make_async_copy(obuf.at[slot], out_hbm.at[pl.ds(0, _CHUNK)],
                              sem_out[slot]).wait()

    _start_in(0)
    for g in range(_NCHUNK):
        _wait_in(g)
        if g + 1 < _NCHUNK:
            _start_in(g + 1)
        if g >= 2:
            _wait_out(g - 2)
        _compute(g % 2)
        _start_out(g)
    _wait_out(_NCHUNK - 2)
    _wait_out(_NCHUNK - 1)
    coord_copy.wait()


def kernel(species, coordinates):
    converted, coords = _convert(species.reshape(_N), coordinates.reshape(_CN))
    return (converted.reshape(species.shape),
            coords.reshape(coordinates.shape))


# speedup vs baseline: 100.5950x; 100.5950x over previous
import functools

import jax
import jax.numpy as jnp
from jax import lax
from jax.experimental import pallas as pl
from jax.experimental.pallas import tpu as pltpu
from jax.experimental.pallas import tpu_sc as plsc

_NC, _NS, _L = 2, 16, 16
_NW = _NC * _NS
_ROWS, _COLS = 16384, 128
_ROWS_W = _ROWS // _NW
_CROWS = 64
_NCHUNK = _ROWS_W // _CROWS
_KPR = _COLS // _L


_GATHER_DNUMS = lax.GatherDimensionNumbers(
    offset_dims=(), collapsed_slice_dims=(0,), start_index_map=(0,))


def _wrap_table16():
    i = lax.iota(jnp.int32, 16)
    return jnp.where(i == jnp.int32(10), jnp.int32(-1),
                     jnp.where(i >= jnp.int32(11), i - jnp.int32(12),
                               i - jnp.int32(1)))


def _map_vec(x, tbl):
    idx = lax.bitwise_and(x, jnp.int32(15))
    return lax.gather(tbl, idx[:, None], _GATHER_DNUMS, (1,),
                      mode=lax.GatherScatterMode.PROMISE_IN_BOUNDS)


@functools.partial(
    pl.kernel,
    mesh=plsc.VectorSubcoreMesh(core_axis_name="c", subcore_axis_name="s"),
    out_type=jax.ShapeDtypeStruct((_ROWS, _COLS), jnp.int32),
    scratch_types=[
        pltpu.VMEM((2, _CROWS, _COLS), jnp.int32),
        pltpu.VMEM((2, _CROWS, _COLS), jnp.int32),
        pltpu.SemaphoreType.DMA,
        pltpu.SemaphoreType.DMA,
        pltpu.SemaphoreType.DMA,
        pltpu.SemaphoreType.DMA,
    ],
)
def _convert(sp_hbm, out_hbm, ibuf, obuf, si0, si1, so0, so1):
    wid = lax.axis_index("s") * _NC + lax.axis_index("c")
    row0 = wid * _ROWS_W
    tbl = _wrap_table16()
    sem_in, sem_out = (si0, si1), (so0, so1)

    def _start_in(g):
        slot = g % 2
        pltpu.async_copy(sp_hbm.at[pl.ds(row0 + g * _CROWS, _CROWS)],
                         ibuf.at[slot], sem_in[slot])

    def _wait_in(g):
        slot = g % 2
        pltpu.make_async_copy(sp_hbm.at[pl.ds(0, _CROWS)], ibuf.at[slot],
                              sem_in[slot]).wait()

    def _compute(slot):
        def body(r, _):
            xs = [ibuf[slot, r, pl.ds(k * _L, _L)] for k in range(_KPR)]
            ys = [_map_vec(x, tbl) for x in xs]
            for k, y in enumerate(ys):
                obuf[slot, r, pl.ds(k * _L, _L)] = y
            return 0
        lax.fori_loop(0, _CROWS, body, 0)

    def _start_out(g):
        slot = g % 2
        pltpu.async_copy(obuf.at[slot],
                         out_hbm.at[pl.ds(row0 + g * _CROWS, _CROWS)],
                         sem_out[slot])

    def _wait_out(g):
        slot = g % 2
        pltpu.make_async_copy(obuf.at[slot], out_hbm.at[pl.ds(0, _CROWS)],
                              sem_out[slot]).wait()

    _start_in(0)
    for g in range(_NCHUNK):
        _wait_in(g)
        if g + 1 < _NCHUNK:
            _start_in(g + 1)
        if g >= 2:
            _wait_out(g - 2)
        _compute(g % 2)
        _start_out(g)
    _wait_out(_NCHUNK - 2)
    _wait_out(_NCHUNK - 1)


def kernel(species, coordinates):
    return (_convert(species), coordinates)


# speedup vs baseline: 120.1797x; 1.1947x over previous
import functools

import jax
import jax.numpy as jnp
from jax import lax
from jax.experimental import pallas as pl
from jax.experimental.pallas import tpu as pltpu
from jax.experimental.pallas import tpu_sc as plsc

_NC, _NS, _L = 2, 16, 16
_NW = _NC * _NS
_ROWS, _COLS = 16384, 128
_ROWS_W = _ROWS // _NW
_CROWS = 64
_NCHUNK = _ROWS_W // _CROWS
_KPR = _COLS // _L


_GATHER_DNUMS = lax.GatherDimensionNumbers(
    offset_dims=(), collapsed_slice_dims=(0,), start_index_map=(0,))


def _wrap_table16():
    i = lax.iota(jnp.int32, 16)
    return jnp.where(i == jnp.int32(10), jnp.int32(-1),
                     jnp.where(i >= jnp.int32(11), i - jnp.int32(12),
                               i - jnp.int32(1)))


def _map_vec(x, tbl):
    idx = lax.bitwise_and(x, jnp.int32(15))
    return lax.gather(tbl, idx[:, None], _GATHER_DNUMS, (1,),
                      mode=lax.GatherScatterMode.PROMISE_IN_BOUNDS)


@functools.partial(
    pl.kernel,
    mesh=plsc.VectorSubcoreMesh(core_axis_name="c", subcore_axis_name="s"),
    out_type=jax.ShapeDtypeStruct((_ROWS, _COLS), jnp.int32),
    scratch_types=[
        pltpu.VMEM((2, _CROWS, _COLS), jnp.int32),
        pltpu.VMEM((2, _CROWS, _COLS), jnp.int32),
        pltpu.SemaphoreType.DMA,
        pltpu.SemaphoreType.DMA,
        pltpu.SemaphoreType.DMA,
        pltpu.SemaphoreType.DMA,
    ],
)
def _convert(sp_hbm, out_hbm, ibuf, obuf, si0, si1, so0, so1):
    wid = lax.axis_index("s") * _NC + lax.axis_index("c")
    row0 = wid * _ROWS_W
    tbl = _wrap_table16()
    sem_in, sem_out = (si0, si1), (so0, so1)

    def _start_in(g):
        slot = g % 2
        pltpu.async_copy(sp_hbm.at[pl.ds(row0 + g * _CROWS, _CROWS)],
                         ibuf.at[slot], sem_in[slot])

    def _wait_in(g):
        slot = g % 2
        pltpu.make_async_copy(sp_hbm.at[pl.ds(0, _CROWS)], ibuf.at[slot],
                              sem_in[slot]).wait()

    def _compute(slot):
        def body(r, _):
            xs = [ibuf[slot, r, pl.ds(k * _L, _L)] for k in range(_KPR)]
            ys = [_map_vec(x, tbl) for x in xs]
            for k, y in enumerate(ys):
                obuf[slot, r, pl.ds(k * _L, _L)] = y
            return 0
        lax.fori_loop(0, _CROWS, body, 0)

    def _start_out(g):
        slot = g % 2
        pltpu.async_copy(obuf.at[slot],
                         out_hbm.at[pl.ds(row0 + g * _CROWS, _CROWS)],
                         sem_out[slot])

    def _wait_out(g):
        slot = g % 2
        pltpu.make_async_copy(obuf.at[slot], out_hbm.at[pl.ds(0, _CROWS)],
                              sem_out[slot]).wait()

    _start_in(0)
    for g in range(_NCHUNK):
        _wait_in(g)
        if g + 1 < _NCHUNK:
            _start_in(g + 1)
        if g >= 2:
            _wait_out(g - 2)
        _compute(g % 2)
        _start_out(g)
    _wait_out(_NCHUNK - 2)
    _wait_out(_NCHUNK - 1)


def kernel(species, coordinates):
    one = lax.optimization_barrier(jnp.float32(1.0))
    return (_convert(species), coordinates * one)


# speedup vs baseline: 125.1680x; 1.0415x over previous
import functools

import jax
import jax.numpy as jnp
from jax import lax
from jax.experimental import pallas as pl
from jax.experimental.pallas import tpu as pltpu
from jax.experimental.pallas import tpu_sc as plsc

_NC, _NS, _L = 2, 16, 16
_NW = _NC * _NS
_ROWS, _COLS = 16384, 128
_ROWS_W = _ROWS // _NW
_CROWS = 64
_NCHUNK = _ROWS_W // _CROWS
_KPR = _COLS // _L


_GATHER_DNUMS = lax.GatherDimensionNumbers(
    offset_dims=(), collapsed_slice_dims=(0,), start_index_map=(0,))


def _wrap_table16():
    i = lax.iota(jnp.int32, 16)
    return jnp.where(i == jnp.int32(10), jnp.int32(-1),
                     jnp.where(i >= jnp.int32(11), i - jnp.int32(12),
                               i - jnp.int32(1)))


def _map_vec(x, tbl):
    idx = lax.bitwise_and(x, jnp.int32(15))
    return lax.gather(tbl, idx[:, None], _GATHER_DNUMS, (1,),
                      mode=lax.GatherScatterMode.PROMISE_IN_BOUNDS)


@functools.partial(
    pl.kernel,
    mesh=plsc.VectorSubcoreMesh(core_axis_name="c", subcore_axis_name="s"),
    out_type=jax.ShapeDtypeStruct((_ROWS, _COLS), jnp.int32),
    scratch_types=[
        pltpu.VMEM((_NCHUNK, _CROWS, _COLS), jnp.int32),
        pltpu.SemaphoreType.DMA,
        pltpu.SemaphoreType.DMA,
    ],
)
def _convert(sp_hbm, out_hbm, buf, sem_in, sem_out):
    wid = lax.axis_index("s") * _NC + lax.axis_index("c")
    row0 = wid * _ROWS_W
    tbl = _wrap_table16()

    def _fire(g, _):
        pltpu.async_copy(sp_hbm.at[pl.ds(row0 + g * _CROWS, _CROWS)],
                         buf.at[g], sem_in)
        return 0
    lax.fori_loop(0, _NCHUNK, _fire, 0)

    def _chunk(g, _):
        pltpu.make_async_copy(sp_hbm.at[pl.ds(0, _CROWS)], buf.at[g],
                              sem_in).wait()

        def body(r, _):
            xs = [buf[g, r, pl.ds(k * _L, _L)] for k in range(_KPR)]
            ys = [_map_vec(x, tbl) for x in xs]
            for k, y in enumerate(ys):
                buf[g, r, pl.ds(k * _L, _L)] = y
            return 0
        lax.fori_loop(0, _CROWS, body, 0)
        pltpu.async_copy(buf.at[g],
                         out_hbm.at[pl.ds(row0 + g * _CROWS, _CROWS)],
                         sem_out)
        return 0
    lax.fori_loop(0, _NCHUNK, _chunk, 0)

    def _drain(g, _):
        pltpu.make_async_copy(buf.at[0], out_hbm.at[pl.ds(0, _CROWS)],
                              sem_out).wait()
        return 0
    lax.fori_loop(0, _NCHUNK, _drain, 0)


def kernel(species, coordinates):
    one = lax.optimization_barrier(jnp.float32(1.0))
    return (_convert(species), coordinates * one)
